# baseline (speedup 1.0000x reference)
import jax
import jax.numpy as jnp
from jax import lax
from jax.experimental import pallas as pl
from jax.experimental.pallas import tpu as pltpu
from jax.experimental.pallas import tpu_sc as plsc

N = 10000
E = 160000
F_IN = 256
HID = 64
HEADS = 8
NCLS = 16

RB = 1000
NT = 16
NP = 10240
NPT = NP // NT
ZR = 128
E2 = 163840

CH1 = 128
EPT1 = E2 // NT
NCH1 = EPT1 // CH1
NG1 = NCH1 // 2

CH2 = 40
EPT2 = E // (2 * NT)
NCH2 = EPT2 // CH2


def _lane_take(v, idx16):
    dnums = lax.GatherDimensionNumbers(
        offset_dims=(), collapsed_slice_dims=(0,), start_index_map=(0,))
    return lax.gather(v, idx16[:, None], dnums, (1,),
                      mode=lax.GatherScatterMode.PROMISE_IN_BOUNDS)


def _mm1_body(x_ref, w1_ref, a1s_ref, a1d_ref, h_ref, als_ref, ald_ref):
    h = jnp.dot(x_ref[...], w1_ref[...], preferred_element_type=jnp.float32)
    h_ref[...] = h
    als_ref[...] = jnp.dot(h, a1s_ref[...], preferred_element_type=jnp.float32)
    ald_ref[...] = jnp.dot(h, a1d_ref[...], preferred_element_type=jnp.float32)


def _stage_a(x, W1, A1s, A1d):
    return pl.pallas_call(
        _mm1_body,
        grid=(N // RB,),
        in_specs=[
            pl.BlockSpec((RB, F_IN), lambda r: (r, 0)),
            pl.BlockSpec((F_IN, HEADS * HID), lambda r: (0, 0)),
            pl.BlockSpec((HEADS * HID, HEADS), lambda r: (0, 0)),
            pl.BlockSpec((HEADS * HID, HEADS), lambda r: (0, 0)),
        ],
        out_specs=[
            pl.BlockSpec((RB, HEADS * HID), lambda r: (r, 0)),
            pl.BlockSpec((RB, HEADS), lambda r: (r, 0)),
            pl.BlockSpec((RB, HEADS), lambda r: (r, 0)),
        ],
        out_shape=[
            jax.ShapeDtypeStruct((N, HEADS * HID), jnp.float32),
            jax.ShapeDtypeStruct((N, HEADS), jnp.float32),
            jax.ShapeDtypeStruct((N, HEADS), jnp.float32),
        ],
    )(x, W1, A1s, A1d)


def _sc1_body(src_hbm, dst_hbm, alS_hbm, alD_hbm, hf_hbm,
              acc_out, den_out,
              src_v, dst_v,
              sA, sB, dA, dB, exA, exB, ixA, ixB, hv, mv,
              zb, zbd, acc0, den_acc,
              smSA, smSB, smDA, smDB, smNA, smNB, smH, smC):
    c = lax.axis_index("c")
    s = lax.axis_index("s")

    BUFS = ((sA, dA, exA, ixA, smSA, smDA, smNA),
            (sB, dB, exB, ixB, smSB, smDB, smNB))

    @pl.loop(0, ZR)
    def _(i):
        for q in range(HID // 16):
            zb[i, pl.ds(q * 16, 16)] = jnp.zeros((16,), jnp.float32)

    @pl.loop(0, NPT)
    def _(i):
        zbd[i, pl.ds(0, 16)] = jnp.zeros((16,), jnp.float32)

    def zero_acc():
        for k in range(NPT // ZR):
            pltpu.sync_copy(zb, acc0.at[pl.ds(s * NPT + k * ZR, ZR)])

    zero_acc()
    pltpu.sync_copy(zbd, den_acc.at[pl.ds(s * NPT, NPT)])

    pltpu.sync_copy(src_hbm.at[s], src_v)
    pltpu.sync_copy(dst_hbm.at[s], dst_v)
    plsc.subcore_barrier()

    for p in range(4):
        hh = 4 * c + p
        hh_splat = jnp.full((16,), hh, jnp.int32)

        def issue(j, b):
            sv, dv, exm, ix, smS, smD, smN = BUFS[b]
            pltpu.async_copy(alS_hbm.at[src_v.at[j]], sv, smS)
            pltpu.async_copy(alD_hbm.at[dst_v.at[j]], dv, smD)

            @pl.loop(0, CH1 // 16)
            def _(k):
                ix[pl.ds(k * 16, 16)] = src_v[j, pl.ds(k * 16, 16)] * 8 + hh

        def process(g, j, b):
            sv, dv, exm, ix, smS, smD, smN = BUFS[b]
            pltpu.make_async_copy(alS_hbm.at[pl.ds(0, CH1)], sv, smS).wait()
            pltpu.make_async_copy(alD_hbm.at[pl.ds(0, CH1)], dv, smD).wait()

            if p == 0:
                @pl.when(g > 0)
                def _():
                    pltpu.make_async_copy(
                        alS_hbm.at[pl.ds(0, CH1)], exm, smN).wait()

            if b == 0:
                @pl.when(g > 0)
                def _():
                    pltpu.make_async_copy(
                        hf_hbm.at[pl.ds(0, CH1)], mv, smC).wait()
            else:
                pltpu.make_async_copy(
                    hf_hbm.at[pl.ds(0, CH1)], mv, smC).wait()

            pltpu.async_copy(hf_hbm.at[ix], hv, smH)

            @pl.loop(0, CH1, unroll=2)
            def _(e):
                a = sv[e, pl.ds(0, 16)] + dv[e, pl.ds(0, 16)]
                exm[e, pl.ds(0, 16)] = jnp.exp(jnp.maximum(a, 0.2 * a))

            didx = dst_v.at[j]
            if p == 0:
                pltpu.async_copy(exm, den_acc.at[didx], smN, add=True)

            pltpu.make_async_copy(hf_hbm.at[pl.ds(0, CH1)], hv, smH).wait()

            @pl.loop(0, CH1, unroll=2)
            def _(e):
                exb = _lane_take(exm[e, pl.ds(0, 16)], hh_splat)
                for q in range(HID // 16):
                    mv[e, pl.ds(q * 16, 16)] = hv[e, pl.ds(q * 16, 16)] * exb

            pltpu.async_copy(mv, acc0.at[didx], smC, add=True)

            @pl.when(g < NG1 - 1)
            def _():
                issue(j + 2, b)

        issue(0, 0)
        issue(1, 1)

        @pl.loop(0, NG1)
        def _(g):
            process(g, 2 * g, 0)
            process(g, 2 * g + 1, 1)

        pltpu.make_async_copy(hf_hbm.at[pl.ds(0, CH1)], mv, smC).wait()
        if p == 0:
            for b in range(2):
                pltpu.make_async_copy(
                    alS_hbm.at[pl.ds(0, CH1)], BUFS[b][2], BUFS[b][6]).wait()

        plsc.subcore_barrier()
        off = s * NPT
        pltpu.sync_copy(acc0.at[pl.ds(off, NPT)],
                        acc_out.at[pl.ds(hh * NP + off, NPT)])
        if p == 0:
            @pl.when(c == 0)
            def _():
                pltpu.sync_copy(den_acc.at[pl.ds(off, NPT)],
                                den_out.at[pl.ds(off, NPT)])
        if p < 3:
            zero_acc()
            plsc.subcore_barrier()


def _stage_b(src2d, dst2d, alS, alD, h_flat):
    mesh = plsc.VectorSubcoreMesh(core_axis_name="c", subcore_axis_name="s")
    kern = pl.kernel(
        _sc1_body,
        mesh=mesh,
        compiler_params=pltpu.CompilerParams(use_tc_tiling_on_sc=False),
        out_type=[
            jax.ShapeDtypeStruct((HEADS * NP, HID), jnp.float32),
            jax.ShapeDtypeStruct((NP, 16), jnp.float32),
        ],
        scratch_types=[
            pltpu.VMEM((NCH1, CH1), jnp.int32),
            pltpu.VMEM((NCH1, CH1), jnp.int32),
            pltpu.VMEM((CH1, 16), jnp.float32),
            pltpu.VMEM((CH1, 16), jnp.float32),
            pltpu.VMEM((CH1, 16), jnp.float32),
            pltpu.VMEM((CH1, 16), jnp.float32),
            pltpu.VMEM((CH1, 16), jnp.float32),
            pltpu.VMEM((CH1, 16), jnp.float32),
            pltpu.VMEM((CH1,), jnp.int32),
            pltpu.VMEM((CH1,), jnp.int32),
            pltpu.VMEM((CH1, HID), jnp.float32),
            pltpu.VMEM((CH1, HID), jnp.float32),
            pltpu.VMEM((ZR, HID), jnp.float32),
            pltpu.VMEM((NPT, 16), jnp.float32),
            pltpu.VMEM_SHARED((NP, HID), jnp.float32),
            pltpu.VMEM_SHARED((NP, 16), jnp.float32),
        ] + [pltpu.SemaphoreType.DMA] * 8,
    )
    return kern(src2d, dst2d, alS, alD, h_flat)


def _mm2_body(acc_ref, den_ref, w2_ref, b1_ref, a2s_ref, a2d_ref, t2_ref):
    h2t = jnp.zeros((RB, NCLS), jnp.float32)
    den = den_ref[...]
    for h in range(HEADS):
        v = acc_ref[h] / (den[:, h][:, None] + 1e-16) + b1_ref[h][None, :]
        v = jnp.where(v > 0, v, jnp.exp(jnp.minimum(v, 0.0)) - 1.0)
        h2t = h2t + jnp.dot(v, w2_ref[h], preferred_element_type=jnp.float32)
    als2 = jnp.sum(h2t * a2s_ref[...], axis=1)
    ald2 = jnp.sum(h2t * a2d_ref[...], axis=1)
    pad = jnp.zeros((RB, 14), jnp.float32)
    t2_ref[...] = jnp.concatenate(
        [h2t, als2[:, None], ald2[:, None], pad], axis=1)


def _stage_c(acc1, den1, W2, b1, a2_src, a2_dst):
    w2r = W2.reshape(HEADS, HID, NCLS)
    b1r = b1.reshape(HEADS, HID)
    return pl.pallas_call(
        _mm2_body,
        grid=(N // RB,),
        in_specs=[
            pl.BlockSpec((HEADS, RB, HID), lambda r: (0, r, 0)),
            pl.BlockSpec((RB, 16), lambda r: (r, 0)),
            pl.BlockSpec((HEADS, HID, NCLS), lambda r: (0, 0, 0)),
            pl.BlockSpec((HEADS, HID), lambda r: (0, 0)),
            pl.BlockSpec((1, NCLS), lambda r: (0, 0)),
            pl.BlockSpec((1, NCLS), lambda r: (0, 0)),
        ],
        out_specs=pl.BlockSpec((RB, 32), lambda r: (r, 0)),
        out_shape=jax.ShapeDtypeStruct((N, 32), jnp.float32),
    )(acc1, den1, w2r, b1r, a2_src, a2_dst)


def _sc2_body(src_hbm, dst_hbm, t2_hbm, acc_out,
              src_v, dst_v, s_rows, d_rows, m_rows, zb, acc2):
    c = lax.axis_index("c")
    s = lax.axis_index("s")
    w = s * 2 + c
    lane = lax.iota(jnp.int32, 16)
    den_mask = lane == 0

    @pl.loop(0, ZR)
    def _(i):
        @pl.loop(0, 2)
        def _(j):
            zb[i, pl.ds(j * 16, 16)] = jnp.zeros((16,), jnp.float32)

    @pl.loop(0, NPT // ZR)
    def _(k):
        pltpu.sync_copy(zb, acc2.at[pl.ds(s * NPT + k * ZR, ZR)])

    pltpu.sync_copy(src_hbm.at[w], src_v)
    pltpu.sync_copy(dst_hbm.at[w], dst_v)
    plsc.subcore_barrier()

    @pl.loop(0, NCH2)
    def _(j):
        sidx = src_v.at[j]
        didx = dst_v.at[j]
        pltpu.sync_copy(t2_hbm.at[sidx], s_rows)
        pltpu.sync_copy(t2_hbm.at[didx], d_rows)

        lane0 = jnp.zeros((16,), jnp.int32)
        lane1 = jnp.ones((16,), jnp.int32)

        @pl.loop(0, CH2)
        def _(e):
            av = (_lane_take(s_rows[e, pl.ds(16, 16)], lane0)
                  + _lane_take(d_rows[e, pl.ds(16, 16)], lane1))
            av = jnp.where(av >= 0, av, 0.2 * av)
            exv = jnp.exp(av)
            m_rows[e, pl.ds(0, 16)] = exv * s_rows[e, pl.ds(0, 16)]
            m_rows[e, pl.ds(16, 16)] = jnp.where(den_mask, exv, 0.0)

        pltpu.sync_copy(m_rows, acc2.at[didx], add=True)

    plsc.subcore_barrier()
    off = s * NPT
    pltpu.sync_copy(acc2.at[pl.ds(off, NPT)],
                    acc_out.at[pl.ds(c * NP + off, NPT)])


def _stage_d(src2d, dst2d, t2):
    mesh = plsc.VectorSubcoreMesh(core_axis_name="c", subcore_axis_name="s")
    kern = pl.kernel(
        _sc2_body,
        mesh=mesh,
        compiler_params=pltpu.CompilerParams(use_tc_tiling_on_sc=False),
        out_type=jax.ShapeDtypeStruct((2 * NP, 32), jnp.float32),
        scratch_types=[
            pltpu.VMEM((NCH2, CH2), jnp.int32),
            pltpu.VMEM((NCH2, CH2), jnp.int32),
            pltpu.VMEM((CH2, 32), jnp.float32),
            pltpu.VMEM((CH2, 32), jnp.float32),
            pltpu.VMEM((CH2, 32), jnp.float32),
            pltpu.VMEM((ZR, 32), jnp.float32),
            pltpu.VMEM_SHARED((NP, 32), jnp.float32),
        ],
    )
    return kern(src2d, dst2d, t2)


def _fin_body(p_ref, b2_ref, o_ref):
    agg = p_ref[0, :, 0:NCLS] + p_ref[1, :, 0:NCLS]
    den = p_ref[0, :, NCLS] + p_ref[1, :, NCLS]
    h2 = agg / (den[:, None] + 1e-16) + b2_ref[...][None, :]
    m = jnp.max(h2, axis=1, keepdims=True)
    sh = h2 - m
    o_ref[...] = sh - jnp.log(jnp.sum(jnp.exp(sh), axis=1, keepdims=True))


def _stage_e(parts, b2):
    return pl.pallas_call(
        _fin_body,
        grid=(N // RB,),
        in_specs=[
            pl.BlockSpec((2, RB, 32), lambda r: (0, r, 0)),
            pl.BlockSpec((NCLS,), lambda r: (0,)),
        ],
        out_specs=pl.BlockSpec((RB, NCLS), lambda r: (r, 0)),
        out_shape=jax.ShapeDtypeStruct((N, NCLS), jnp.float32),
    )(parts, b2)


def kernel(x, edge_index, W1, a1_src, a1_dst, b1, W2, a2_src, a2_dst, b2):
    eye = jnp.eye(HEADS, dtype=jnp.float32)
    A1s = (eye[:, None, :] * a1_src[:, :, None]).reshape(HEADS * HID, HEADS)
    A1d = (eye[:, None, :] * a1_dst[:, :, None]).reshape(HEADS * HID, HEADS)

    h_all, als, ald = _stage_a(x, W1, A1s, A1d)
    alS = jnp.concatenate([als, ald], axis=1)
    alD = jnp.concatenate([ald, als], axis=1)
    h_flat = h_all.reshape(N * HEADS, HID)

    npad = E2 - E
    src_p = jnp.concatenate([edge_index[0], jnp.zeros((npad,), jnp.int32)])
    dst_p = jnp.concatenate(
        [edge_index[1], jnp.full((npad,), NP - 1, jnp.int32)])

    src1 = src_p.reshape(NT, NCH1, CH1)
    dst1 = dst_p.reshape(NT, NCH1, CH1)
    acc1, den1 = _stage_b(src1, dst1, alS, alD, h_flat)
    acc1 = acc1.reshape(HEADS, NP, HID)[:, :N]
    den1 = den1[:N]

    t2 = _stage_c(acc1, den1, W2, b1, a2_src, a2_dst)

    src2 = edge_index[0].reshape(2 * NT, NCH2, CH2)
    dst2 = edge_index[1].reshape(2 * NT, NCH2, CH2)
    parts = _stage_d(src2, dst2, t2).reshape(2, NP, 32)[:, :N]

    return _stage_e(parts, b2)


# speedup vs baseline: 1.4446x; 1.4446x over previous
import jax
import jax.numpy as jnp
from jax import lax
from jax.experimental import pallas as pl
from jax.experimental.pallas import tpu as pltpu
from jax.experimental.pallas import tpu_sc as plsc

N = 10000
E = 160000
F_IN = 256
HID = 64
HEADS = 8
NCLS = 16

RB = 1000
NT = 16
NP = 10240
NPT = NP // NT
ZR = 128
E2 = 163840

CH1 = 128
EPT1 = E2 // NT
NCH1 = EPT1 // CH1
NG1 = NCH1 // 2

CH2 = 40
EPT2 = E // (2 * NT)
NCH2 = EPT2 // CH2


def _lane_take(v, idx16):
    dnums = lax.GatherDimensionNumbers(
        offset_dims=(), collapsed_slice_dims=(0,), start_index_map=(0,))
    return lax.gather(v, idx16[:, None], dnums, (1,),
                      mode=lax.GatherScatterMode.PROMISE_IN_BOUNDS)


def _mm1_body(x_ref, w1_ref, a1s_ref, a1d_ref, h_ref, als_ref, ald_ref):
    h = jnp.dot(x_ref[...], w1_ref[...], preferred_element_type=jnp.float32)
    h_ref[...] = h
    als_ref[...] = jnp.dot(h, a1s_ref[...], preferred_element_type=jnp.float32)
    ald_ref[...] = jnp.dot(h, a1d_ref[...], preferred_element_type=jnp.float32)


def _stage_a(x, W1, A1s, A1d):
    return pl.pallas_call(
        _mm1_body,
        grid=(N // RB,),
        in_specs=[
            pl.BlockSpec((RB, F_IN), lambda r: (r, 0)),
            pl.BlockSpec((F_IN, HEADS * HID), lambda r: (0, 0)),
            pl.BlockSpec((HEADS * HID, HEADS), lambda r: (0, 0)),
            pl.BlockSpec((HEADS * HID, HEADS), lambda r: (0, 0)),
        ],
        out_specs=[
            pl.BlockSpec((RB, HEADS * HID), lambda r: (r, 0)),
            pl.BlockSpec((RB, HEADS), lambda r: (r, 0)),
            pl.BlockSpec((RB, HEADS), lambda r: (r, 0)),
        ],
        out_shape=[
            jax.ShapeDtypeStruct((N, HEADS * HID), jnp.float32),
            jax.ShapeDtypeStruct((N, HEADS), jnp.float32),
            jax.ShapeDtypeStruct((N, HEADS), jnp.float32),
        ],
    )(x, W1, A1s, A1d)


def _sc1_body(src_hbm, dst_hbm, alS_hbm, alD_hbm, hf_hbm,
              acc_out, den_out,
              src_v, dst_v,
              sA, sB, dA, dB, exA, exB, ixA, ixB, hv, mv,
              zb, zbd, acc0, den_acc,
              smSA, smSB, smDA, smDB, smNA, smNB, smH, smC):
    c = lax.axis_index("c")
    s = lax.axis_index("s")

    BUFS = ((sA, dA, exA, ixA, smSA, smDA, smNA),
            (sB, dB, exB, ixB, smSB, smDB, smNB))

    @pl.loop(0, ZR)
    def _(i):
        for q in range(HID // 16):
            zb[i, pl.ds(q * 16, 16)] = jnp.zeros((16,), jnp.float32)

    @pl.loop(0, NPT)
    def _(i):
        zbd[i, pl.ds(0, 16)] = jnp.zeros((16,), jnp.float32)

    def zero_acc():
        for k in range(NPT // ZR):
            pltpu.sync_copy(zb, acc0.at[pl.ds(s * NPT + k * ZR, ZR)])

    zero_acc()
    pltpu.sync_copy(zbd, den_acc.at[pl.ds(s * NPT, NPT)])

    pltpu.sync_copy(src_hbm.at[s], src_v)
    pltpu.sync_copy(dst_hbm.at[s], dst_v)
    plsc.subcore_barrier()

    for p in range(4):
        hh = 4 * c + p
        hh_splat = jnp.full((16,), hh, jnp.int32)

        def issue(j, b):
            sv, dv, exm, ix, smS, smD, smN = BUFS[b]
            pltpu.async_copy(alS_hbm.at[src_v.at[j]], sv, smS)
            pltpu.async_copy(alD_hbm.at[dst_v.at[j]], dv, smD)

            @pl.loop(0, CH1 // 16)
            def _(k):
                ix[pl.ds(k * 16, 16)] = src_v[j, pl.ds(k * 16, 16)] * 8 + hh

        def process(g, j, b):
            sv, dv, exm, ix, smS, smD, smN = BUFS[b]
            pltpu.make_async_copy(alS_hbm.at[pl.ds(0, CH1)], sv, smS).wait()
            pltpu.make_async_copy(alD_hbm.at[pl.ds(0, CH1)], dv, smD).wait()

            if p == 0:
                @pl.when(g > 0)
                def _():
                    pltpu.make_async_copy(
                        alS_hbm.at[pl.ds(0, CH1)], exm, smN).wait()

            if b == 0:
                @pl.when(g > 0)
                def _():
                    pltpu.make_async_copy(
                        hf_hbm.at[pl.ds(0, CH1)], mv, smC).wait()
            else:
                pltpu.make_async_copy(
                    hf_hbm.at[pl.ds(0, CH1)], mv, smC).wait()

            pltpu.async_copy(hf_hbm.at[ix], hv, smH)

            @pl.loop(0, CH1)
            def _(e):
                a = sv[e, pl.ds(0, 16)] + dv[e, pl.ds(0, 16)]
                exm[e, pl.ds(0, 16)] = jnp.exp(jnp.maximum(a, 0.2 * a))

            didx = dst_v.at[j]
            if p == 0:
                pltpu.async_copy(exm, den_acc.at[didx], smN, add=True)

            pltpu.make_async_copy(hf_hbm.at[pl.ds(0, CH1)], hv, smH).wait()

            @pl.loop(0, CH1)
            def _(e):
                exb = _lane_take(exm[e, pl.ds(0, 16)], hh_splat)
                for q in range(HID // 16):
                    mv[e, pl.ds(q * 16, 16)] = hv[e, pl.ds(q * 16, 16)] * exb

            pltpu.async_copy(mv, acc0.at[didx], smC, add=True)

            @pl.when(g < NG1 - 1)
            def _():
                issue(j + 2, b)

        issue(0, 0)
        issue(1, 1)

        @pl.loop(0, NG1)
        def _(g):
            process(g, 2 * g, 0)
            process(g, 2 * g + 1, 1)

        pltpu.make_async_copy(hf_hbm.at[pl.ds(0, CH1)], mv, smC).wait()
        if p == 0:
            for b in range(2):
                pltpu.make_async_copy(
                    alS_hbm.at[pl.ds(0, CH1)], BUFS[b][2], BUFS[b][6]).wait()

        plsc.subcore_barrier()
        off = s * NPT
        pltpu.sync_copy(acc0.at[pl.ds(off, NPT)],
                        acc_out.at[pl.ds(hh * NP + off, NPT)])
        if p == 0:
            @pl.when(c == 0)
            def _():
                pltpu.sync_copy(den_acc.at[pl.ds(off, NPT)],
                                den_out.at[pl.ds(off, NPT)])
        if p < 3:
            zero_acc()
            plsc.subcore_barrier()


def _stage_b(src2d, dst2d, alS, alD, h_flat):
    mesh = plsc.VectorSubcoreMesh(core_axis_name="c", subcore_axis_name="s")
    kern = pl.kernel(
        _sc1_body,
        mesh=mesh,
        compiler_params=pltpu.CompilerParams(use_tc_tiling_on_sc=False),
        out_type=[
            jax.ShapeDtypeStruct((HEADS * NP, HID), jnp.float32),
            jax.ShapeDtypeStruct((NP, 16), jnp.float32),
        ],
        scratch_types=[
            pltpu.VMEM((NCH1, CH1), jnp.int32),
            pltpu.VMEM((NCH1, CH1), jnp.int32),
            pltpu.VMEM((CH1, 16), jnp.float32),
            pltpu.VMEM((CH1, 16), jnp.float32),
            pltpu.VMEM((CH1, 16), jnp.float32),
            pltpu.VMEM((CH1, 16), jnp.float32),
            pltpu.VMEM((CH1, 16), jnp.float32),
            pltpu.VMEM((CH1, 16), jnp.float32),
            pltpu.VMEM((CH1,), jnp.int32),
            pltpu.VMEM((CH1,), jnp.int32),
            pltpu.VMEM((CH1, HID), jnp.float32),
            pltpu.VMEM((CH1, HID), jnp.float32),
            pltpu.VMEM((ZR, HID), jnp.float32),
            pltpu.VMEM((NPT, 16), jnp.float32),
            pltpu.VMEM_SHARED((NP, HID), jnp.float32),
            pltpu.VMEM_SHARED((NP, 16), jnp.float32),
        ] + [pltpu.SemaphoreType.DMA] * 8,
    )
    return kern(src2d, dst2d, alS, alD, h_flat)


def _mm2_body(acc_ref, den_ref, w2_ref, b1_ref, a2s_ref, a2d_ref, t2_ref):
    h2t = jnp.zeros((RB, NCLS), jnp.float32)
    den = den_ref[...]
    for h in range(HEADS):
        v = acc_ref[h] / (den[:, h][:, None] + 1e-16) + b1_ref[h][None, :]
        v = jnp.where(v > 0, v, jnp.exp(jnp.minimum(v, 0.0)) - 1.0)
        h2t = h2t + jnp.dot(v, w2_ref[h], preferred_element_type=jnp.float32)
    als2 = jnp.sum(h2t * a2s_ref[...], axis=1)
    ald2 = jnp.sum(h2t * a2d_ref[...], axis=1)
    pad = jnp.zeros((RB, 14), jnp.float32)
    t2_ref[...] = jnp.concatenate(
        [h2t, als2[:, None], ald2[:, None], pad], axis=1)


def _stage_c(acc1, den1, W2, b1, a2_src, a2_dst):
    w2r = W2.reshape(HEADS, HID, NCLS)
    b1r = b1.reshape(HEADS, HID)
    return pl.pallas_call(
        _mm2_body,
        grid=(N // RB,),
        in_specs=[
            pl.BlockSpec((HEADS, RB, HID), lambda r: (0, r, 0)),
            pl.BlockSpec((RB, 16), lambda r: (r, 0)),
            pl.BlockSpec((HEADS, HID, NCLS), lambda r: (0, 0, 0)),
            pl.BlockSpec((HEADS, HID), lambda r: (0, 0)),
            pl.BlockSpec((1, NCLS), lambda r: (0, 0)),
            pl.BlockSpec((1, NCLS), lambda r: (0, 0)),
        ],
        out_specs=pl.BlockSpec((RB, 32), lambda r: (r, 0)),
        out_shape=jax.ShapeDtypeStruct((N, 32), jnp.float32),
    )(acc1, den1, w2r, b1r, a2_src, a2_dst)


def _sc2_body(src_hbm, dst_hbm, t2_hbm, acc_out,
              src_v, dst_v, s_rows, d_rows, m_rows, zb, acc2):
    c = lax.axis_index("c")
    s = lax.axis_index("s")
    w = s * 2 + c
    lane = lax.iota(jnp.int32, 16)
    den_mask = lane == 0

    @pl.loop(0, ZR)
    def _(i):
        @pl.loop(0, 2)
        def _(j):
            zb[i, pl.ds(j * 16, 16)] = jnp.zeros((16,), jnp.float32)

    @pl.loop(0, NPT // ZR)
    def _(k):
        pltpu.sync_copy(zb, acc2.at[pl.ds(s * NPT + k * ZR, ZR)])

    pltpu.sync_copy(src_hbm.at[w], src_v)
    pltpu.sync_copy(dst_hbm.at[w], dst_v)
    plsc.subcore_barrier()

    @pl.loop(0, NCH2)
    def _(j):
        sidx = src_v.at[j]
        didx = dst_v.at[j]
        pltpu.sync_copy(t2_hbm.at[sidx], s_rows)
        pltpu.sync_copy(t2_hbm.at[didx], d_rows)

        lane0 = jnp.zeros((16,), jnp.int32)
        lane1 = jnp.ones((16,), jnp.int32)

        @pl.loop(0, CH2)
        def _(e):
            av = (_lane_take(s_rows[e, pl.ds(16, 16)], lane0)
                  + _lane_take(d_rows[e, pl.ds(16, 16)], lane1))
            av = jnp.where(av >= 0, av, 0.2 * av)
            exv = jnp.exp(av)
            m_rows[e, pl.ds(0, 16)] = exv * s_rows[e, pl.ds(0, 16)]
            m_rows[e, pl.ds(16, 16)] = jnp.where(den_mask, exv, 0.0)

        pltpu.sync_copy(m_rows, acc2.at[didx], add=True)

    plsc.subcore_barrier()
    off = s * NPT
    pltpu.sync_copy(acc2.at[pl.ds(off, NPT)],
                    acc_out.at[pl.ds(c * NP + off, NPT)])


def _stage_d(src2d, dst2d, t2):
    mesh = plsc.VectorSubcoreMesh(core_axis_name="c", subcore_axis_name="s")
    kern = pl.kernel(
        _sc2_body,
        mesh=mesh,
        compiler_params=pltpu.CompilerParams(use_tc_tiling_on_sc=False),
        out_type=jax.ShapeDtypeStruct((2 * NP, 32), jnp.float32),
        scratch_types=[
            pltpu.VMEM((NCH2, CH2), jnp.int32),
            pltpu.VMEM((NCH2, CH2), jnp.int32),
            pltpu.VMEM((CH2, 32), jnp.float32),
            pltpu.VMEM((CH2, 32), jnp.float32),
            pltpu.VMEM((CH2, 32), jnp.float32),
            pltpu.VMEM((ZR, 32), jnp.float32),
            pltpu.VMEM_SHARED((NP, 32), jnp.float32),
        ],
    )
    return kern(src2d, dst2d, t2)


def _fin_body(p_ref, b2_ref, o_ref):
    agg = p_ref[0, :, 0:NCLS] + p_ref[1, :, 0:NCLS]
    den = p_ref[0, :, NCLS] + p_ref[1, :, NCLS]
    h2 = agg / (den[:, None] + 1e-16) + b2_ref[...][None, :]
    m = jnp.max(h2, axis=1, keepdims=True)
    sh = h2 - m
    o_ref[...] = sh - jnp.log(jnp.sum(jnp.exp(sh), axis=1, keepdims=True))


def _stage_e(parts, b2):
    return pl.pallas_call(
        _fin_body,
        grid=(N // RB,),
        in_specs=[
            pl.BlockSpec((2, RB, 32), lambda r: (0, r, 0)),
            pl.BlockSpec((NCLS,), lambda r: (0,)),
        ],
        out_specs=pl.BlockSpec((RB, NCLS), lambda r: (r, 0)),
        out_shape=jax.ShapeDtypeStruct((N, NCLS), jnp.float32),
    )(parts, b2)


def kernel(x, edge_index, W1, a1_src, a1_dst, b1, W2, a2_src, a2_dst, b2):
    eye = jnp.eye(HEADS, dtype=jnp.float32)
    A1s = (eye[:, None, :] * a1_src[:, :, None]).reshape(HEADS * HID, HEADS)
    A1d = (eye[:, None, :] * a1_dst[:, :, None]).reshape(HEADS * HID, HEADS)

    h_all, als, ald = _stage_a(x, W1, A1s, A1d)
    alS = jnp.concatenate([als, ald], axis=1)
    alD = jnp.concatenate([ald, als], axis=1)
    h_flat = h_all.reshape(N * HEADS, HID)

    npad = E2 - E
    src_p = jnp.concatenate([edge_index[0], jnp.zeros((npad,), jnp.int32)])
    dst_p = jnp.concatenate(
        [edge_index[1], jnp.full((npad,), NP - 1, jnp.int32)])

    src1 = src_p.reshape(NT, NCH1, CH1)
    dst1 = dst_p.reshape(NT, NCH1, CH1)
    acc1, den1 = _stage_b(src1, dst1, alS, alD, h_flat)
    acc1 = acc1.reshape(HEADS, NP, HID)[:, :N]
    den1 = den1[:N]

    t2 = _stage_c(acc1, den1, W2, b1, a2_src, a2_dst)

    src2 = edge_index[0].reshape(2 * NT, NCH2, CH2)
    dst2 = edge_index[1].reshape(2 * NT, NCH2, CH2)
    parts = _stage_d(src2, dst2, t2).reshape(2, NP, 32)[:, :N]

    return _stage_e(parts, b2)


# speedup vs baseline: 1.4708x; 1.0181x over previous
import jax
import jax.numpy as jnp
from jax import lax
from jax.experimental import pallas as pl
from jax.experimental.pallas import tpu as pltpu
from jax.experimental.pallas import tpu_sc as plsc

N = 10000
E = 160000
F_IN = 256
HID = 64
HEADS = 8
NCLS = 16

RB = 1000
NT = 16
NP = 10240
NPT = NP // NT
ZR = 128
E2 = 163840

CH1 = 128
EPT1 = E2 // NT
NCH1 = EPT1 // CH1
NG1 = NCH1 // 2

CH2 = 40
EPT2 = E // (2 * NT)
NCH2 = EPT2 // CH2


def _lane_take(v, idx16):
    dnums = lax.GatherDimensionNumbers(
        offset_dims=(), collapsed_slice_dims=(0,), start_index_map=(0,))
    return lax.gather(v, idx16[:, None], dnums, (1,),
                      mode=lax.GatherScatterMode.PROMISE_IN_BOUNDS)


def _mm1_body(x_ref, w1_ref, a1s_ref, a1d_ref, h_ref, als_ref, ald_ref):
    h = jnp.dot(x_ref[...], w1_ref[...], preferred_element_type=jnp.float32)
    h_ref[...] = h
    als_ref[...] = jnp.dot(h, a1s_ref[...], preferred_element_type=jnp.float32)
    ald_ref[...] = jnp.dot(h, a1d_ref[...], preferred_element_type=jnp.float32)


def _stage_a(x, W1, A1s, A1d):
    return pl.pallas_call(
        _mm1_body,
        grid=(N // RB,),
        in_specs=[
            pl.BlockSpec((RB, F_IN), lambda r: (r, 0)),
            pl.BlockSpec((F_IN, HEADS * HID), lambda r: (0, 0)),
            pl.BlockSpec((HEADS * HID, HEADS), lambda r: (0, 0)),
            pl.BlockSpec((HEADS * HID, HEADS), lambda r: (0, 0)),
        ],
        out_specs=[
            pl.BlockSpec((RB, HEADS * HID), lambda r: (r, 0)),
            pl.BlockSpec((RB, HEADS), lambda r: (r, 0)),
            pl.BlockSpec((RB, HEADS), lambda r: (r, 0)),
        ],
        out_shape=[
            jax.ShapeDtypeStruct((N, HEADS * HID), jnp.float32),
            jax.ShapeDtypeStruct((N, HEADS), jnp.float32),
            jax.ShapeDtypeStruct((N, HEADS), jnp.float32),
        ],
    )(x, W1, A1s, A1d)


def _sc1_body(src_hbm, dst_hbm, alS_hbm, alD_hbm, hf_hbm,
              acc_out, den_out, ex_hbm,
              src_v, dst_v,
              sA, sB, dA, dB, exA, exB, ixA, ixB, hv, mv,
              zb, zbd, acc0, den_acc,
              smSA, smSB, smDA, smDB, smNA, smNB, smH, smC,
              smWA, smWB):
    c = lax.axis_index("c")
    s = lax.axis_index("s")

    BUFS = ((sA, dA, exA, ixA, smSA, smDA, smNA, smWA),
            (sB, dB, exB, ixB, smSB, smDB, smNB, smWB))

    @pl.loop(0, ZR)
    def _(i):
        for q in range(HID // 16):
            zb[i, pl.ds(q * 16, 16)] = jnp.zeros((16,), jnp.float32)

    @pl.loop(0, NPT)
    def _(i):
        zbd[i, pl.ds(0, 16)] = jnp.zeros((16,), jnp.float32)

    def zero_acc():
        for k in range(NPT // ZR):
            pltpu.sync_copy(zb, acc0.at[pl.ds(s * NPT + k * ZR, ZR)])

    zero_acc()
    pltpu.sync_copy(zbd, den_acc.at[pl.ds(s * NPT, NPT)])

    pltpu.sync_copy(src_hbm.at[s], src_v)
    pltpu.sync_copy(dst_hbm.at[s], dst_v)
    plsc.subcore_barrier()

    for p in range(4):
        hh = 4 * c + p
        hh_splat = jnp.full((16,), hh, jnp.int32)

        exbase = c * E2 + s * EPT1

        def issue(j, b):
            sv, dv, exm, ix, smS, smD, smN, smW = BUFS[b]
            if p == 0:
                pltpu.async_copy(alS_hbm.at[src_v.at[j]], sv, smS)
                pltpu.async_copy(alD_hbm.at[dst_v.at[j]], dv, smD)
            else:
                pltpu.async_copy(
                    ex_hbm.at[pl.ds(exbase + j * CH1, CH1)], exm, smS)

            @pl.loop(0, CH1 // 16)
            def _(k):
                ix[pl.ds(k * 16, 16)] = src_v[j, pl.ds(k * 16, 16)] * 8 + hh

        def process(g, j, b):
            sv, dv, exm, ix, smS, smD, smN, smW = BUFS[b]
            if p == 0:
                pltpu.make_async_copy(
                    alS_hbm.at[pl.ds(0, CH1)], sv, smS).wait()
                pltpu.make_async_copy(
                    alD_hbm.at[pl.ds(0, CH1)], dv, smD).wait()

                @pl.when(g > 0)
                def _():
                    pltpu.make_async_copy(
                        alS_hbm.at[pl.ds(0, CH1)], exm, smN).wait()
                    pltpu.make_async_copy(
                        alS_hbm.at[pl.ds(0, CH1)], exm, smW).wait()
            else:
                pltpu.make_async_copy(
                    alS_hbm.at[pl.ds(0, CH1)], exm, smS).wait()

            if b == 0:
                @pl.when(g > 0)
                def _():
                    pltpu.make_async_copy(
                        hf_hbm.at[pl.ds(0, CH1)], mv, smC).wait()
            else:
                pltpu.make_async_copy(
                    hf_hbm.at[pl.ds(0, CH1)], mv, smC).wait()

            pltpu.async_copy(hf_hbm.at[ix], hv, smH)

            didx = dst_v.at[j]
            if p == 0:
                @pl.loop(0, CH1)
                def _(e):
                    a = sv[e, pl.ds(0, 16)] + dv[e, pl.ds(0, 16)]
                    exm[e, pl.ds(0, 16)] = jnp.exp(jnp.maximum(a, 0.2 * a))

                pltpu.async_copy(exm, den_acc.at[didx], smN, add=True)
                pltpu.async_copy(
                    exm, ex_hbm.at[pl.ds(exbase + j * CH1, CH1)], smW)

            pltpu.make_async_copy(hf_hbm.at[pl.ds(0, CH1)], hv, smH).wait()

            @pl.loop(0, CH1)
            def _(e):
                exb = _lane_take(exm[e, pl.ds(0, 16)], hh_splat)
                for q in range(HID // 16):
                    mv[e, pl.ds(q * 16, 16)] = hv[e, pl.ds(q * 16, 16)] * exb

            pltpu.async_copy(mv, acc0.at[didx], smC, add=True)

            @pl.when(g < NG1 - 1)
            def _():
                issue(j + 2, b)

        issue(0, 0)
        issue(1, 1)

        @pl.loop(0, NG1)
        def _(g):
            process(g, 2 * g, 0)
            process(g, 2 * g + 1, 1)

        pltpu.make_async_copy(hf_hbm.at[pl.ds(0, CH1)], mv, smC).wait()
        if p == 0:
            for b in range(2):
                pltpu.make_async_copy(
                    alS_hbm.at[pl.ds(0, CH1)], BUFS[b][2], BUFS[b][6]).wait()
                pltpu.make_async_copy(
                    alS_hbm.at[pl.ds(0, CH1)], BUFS[b][2], BUFS[b][7]).wait()

        plsc.subcore_barrier()
        off = s * NPT
        pltpu.sync_copy(acc0.at[pl.ds(off, NPT)],
                        acc_out.at[pl.ds(hh * NP + off, NPT)])
        if p == 0:
            @pl.when(c == 0)
            def _():
                pltpu.sync_copy(den_acc.at[pl.ds(off, NPT)],
                                den_out.at[pl.ds(off, NPT)])
        if p < 3:
            zero_acc()
            plsc.subcore_barrier()


def _stage_b(src2d, dst2d, alS, alD, h_flat):
    mesh = plsc.VectorSubcoreMesh(core_axis_name="c", subcore_axis_name="s")
    kern = pl.kernel(
        _sc1_body,
        mesh=mesh,
        compiler_params=pltpu.CompilerParams(use_tc_tiling_on_sc=False),
        out_type=[
            jax.ShapeDtypeStruct((HEADS * NP, HID), jnp.float32),
            jax.ShapeDtypeStruct((NP, 16), jnp.float32),
            jax.ShapeDtypeStruct((2 * E2, 16), jnp.float32),
        ],
        scratch_types=[
            pltpu.VMEM((NCH1, CH1), jnp.int32),
            pltpu.VMEM((NCH1, CH1), jnp.int32),
            pltpu.VMEM((CH1, 16), jnp.float32),
            pltpu.VMEM((CH1, 16), jnp.float32),
            pltpu.VMEM((CH1, 16), jnp.float32),
            pltpu.VMEM((CH1, 16), jnp.float32),
            pltpu.VMEM((CH1, 16), jnp.float32),
            pltpu.VMEM((CH1, 16), jnp.float32),
            pltpu.VMEM((CH1,), jnp.int32),
            pltpu.VMEM((CH1,), jnp.int32),
            pltpu.VMEM((CH1, HID), jnp.float32),
            pltpu.VMEM((CH1, HID), jnp.float32),
            pltpu.VMEM((ZR, HID), jnp.float32),
            pltpu.VMEM((NPT, 16), jnp.float32),
            pltpu.VMEM_SHARED((NP, HID), jnp.float32),
            pltpu.VMEM_SHARED((NP, 16), jnp.float32),
        ] + [pltpu.SemaphoreType.DMA] * 10,
    )
    return kern(src2d, dst2d, alS, alD, h_flat)


def _mm2_body(acc_ref, den_ref, w2_ref, b1_ref, a2s_ref, a2d_ref, t2_ref):
    h2t = jnp.zeros((RB, NCLS), jnp.float32)
    den = den_ref[...]
    for h in range(HEADS):
        v = acc_ref[h] / (den[:, h][:, None] + 1e-16) + b1_ref[h][None, :]
        v = jnp.where(v > 0, v, jnp.exp(jnp.minimum(v, 0.0)) - 1.0)
        h2t = h2t + jnp.dot(v, w2_ref[h], preferred_element_type=jnp.float32)
    als2 = jnp.sum(h2t * a2s_ref[...], axis=1)
    ald2 = jnp.sum(h2t * a2d_ref[...], axis=1)
    pad = jnp.zeros((RB, 14), jnp.float32)
    t2_ref[...] = jnp.concatenate(
        [h2t, als2[:, None], ald2[:, None], pad], axis=1)


def _stage_c(acc1, den1, W2, b1, a2_src, a2_dst):
    w2r = W2.reshape(HEADS, HID, NCLS)
    b1r = b1.reshape(HEADS, HID)
    return pl.pallas_call(
        _mm2_body,
        grid=(N // RB,),
        in_specs=[
            pl.BlockSpec((HEADS, RB, HID), lambda r: (0, r, 0)),
            pl.BlockSpec((RB, 16), lambda r: (r, 0)),
            pl.BlockSpec((HEADS, HID, NCLS), lambda r: (0, 0, 0)),
            pl.BlockSpec((HEADS, HID), lambda r: (0, 0)),
            pl.BlockSpec((1, NCLS), lambda r: (0, 0)),
            pl.BlockSpec((1, NCLS), lambda r: (0, 0)),
        ],
        out_specs=pl.BlockSpec((RB, 32), lambda r: (r, 0)),
        out_shape=jax.ShapeDtypeStruct((N, 32), jnp.float32),
    )(acc1, den1, w2r, b1r, a2_src, a2_dst)


def _sc2_body(src_hbm, dst_hbm, t2_hbm, acc_out,
              src_v, dst_v, s_rows, d_rows, m_rows, zb, acc2):
    c = lax.axis_index("c")
    s = lax.axis_index("s")
    w = s * 2 + c
    lane = lax.iota(jnp.int32, 16)
    den_mask = lane == 0

    @pl.loop(0, ZR)
    def _(i):
        @pl.loop(0, 2)
        def _(j):
            zb[i, pl.ds(j * 16, 16)] = jnp.zeros((16,), jnp.float32)

    @pl.loop(0, NPT // ZR)
    def _(k):
        pltpu.sync_copy(zb, acc2.at[pl.ds(s * NPT + k * ZR, ZR)])

    pltpu.sync_copy(src_hbm.at[w], src_v)
    pltpu.sync_copy(dst_hbm.at[w], dst_v)
    plsc.subcore_barrier()

    @pl.loop(0, NCH2)
    def _(j):
        sidx = src_v.at[j]
        didx = dst_v.at[j]
        pltpu.sync_copy(t2_hbm.at[sidx], s_rows)
        pltpu.sync_copy(t2_hbm.at[didx], d_rows)

        lane0 = jnp.zeros((16,), jnp.int32)
        lane1 = jnp.ones((16,), jnp.int32)

        @pl.loop(0, CH2)
        def _(e):
            av = (_lane_take(s_rows[e, pl.ds(16, 16)], lane0)
                  + _lane_take(d_rows[e, pl.ds(16, 16)], lane1))
            av = jnp.where(av >= 0, av, 0.2 * av)
            exv = jnp.exp(av)
            m_rows[e, pl.ds(0, 16)] = exv * s_rows[e, pl.ds(0, 16)]
            m_rows[e, pl.ds(16, 16)] = jnp.where(den_mask, exv, 0.0)

        pltpu.sync_copy(m_rows, acc2.at[didx], add=True)

    plsc.subcore_barrier()
    off = s * NPT
    pltpu.sync_copy(acc2.at[pl.ds(off, NPT)],
                    acc_out.at[pl.ds(c * NP + off, NPT)])


def _stage_d(src2d, dst2d, t2):
    mesh = plsc.VectorSubcoreMesh(core_axis_name="c", subcore_axis_name="s")
    kern = pl.kernel(
        _sc2_body,
        mesh=mesh,
        compiler_params=pltpu.CompilerParams(use_tc_tiling_on_sc=False),
        out_type=jax.ShapeDtypeStruct((2 * NP, 32), jnp.float32),
        scratch_types=[
            pltpu.VMEM((NCH2, CH2), jnp.int32),
            pltpu.VMEM((NCH2, CH2), jnp.int32),
            pltpu.VMEM((CH2, 32), jnp.float32),
            pltpu.VMEM((CH2, 32), jnp.float32),
            pltpu.VMEM((CH2, 32), jnp.float32),
            pltpu.VMEM((ZR, 32), jnp.float32),
            pltpu.VMEM_SHARED((NP, 32), jnp.float32),
        ],
    )
    return kern(src2d, dst2d, t2)


def _fin_body(p_ref, b2_ref, o_ref):
    agg = p_ref[0, :, 0:NCLS] + p_ref[1, :, 0:NCLS]
    den = p_ref[0, :, NCLS] + p_ref[1, :, NCLS]
    h2 = agg / (den[:, None] + 1e-16) + b2_ref[...][None, :]
    m = jnp.max(h2, axis=1, keepdims=True)
    sh = h2 - m
    o_ref[...] = sh - jnp.log(jnp.sum(jnp.exp(sh), axis=1, keepdims=True))


def _stage_e(parts, b2):
    return pl.pallas_call(
        _fin_body,
        grid=(N // RB,),
        in_specs=[
            pl.BlockSpec((2, RB, 32), lambda r: (0, r, 0)),
            pl.BlockSpec((NCLS,), lambda r: (0,)),
        ],
        out_specs=pl.BlockSpec((RB, NCLS), lambda r: (r, 0)),
        out_shape=jax.ShapeDtypeStruct((N, NCLS), jnp.float32),
    )(parts, b2)


def kernel(x, edge_index, W1, a1_src, a1_dst, b1, W2, a2_src, a2_dst, b2):
    eye = jnp.eye(HEADS, dtype=jnp.float32)
    A1s = (eye[:, None, :] * a1_src[:, :, None]).reshape(HEADS * HID, HEADS)
    A1d = (eye[:, None, :] * a1_dst[:, :, None]).reshape(HEADS * HID, HEADS)

    h_all, als, ald = _stage_a(x, W1, A1s, A1d)
    alS = jnp.concatenate([als, ald], axis=1)
    alD = jnp.concatenate([ald, als], axis=1)
    h_flat = h_all.reshape(N * HEADS, HID)

    npad = E2 - E
    src_p = jnp.concatenate([edge_index[0], jnp.zeros((npad,), jnp.int32)])
    dst_p = jnp.concatenate(
        [edge_index[1], jnp.full((npad,), NP - 1, jnp.int32)])

    src1 = src_p.reshape(NT, NCH1, CH1)
    dst1 = dst_p.reshape(NT, NCH1, CH1)
    acc1, den1, _excache = _stage_b(src1, dst1, alS, alD, h_flat)
    acc1 = acc1.reshape(HEADS, NP, HID)[:, :N]
    den1 = den1[:N]

    t2 = _stage_c(acc1, den1, W2, b1, a2_src, a2_dst)

    src2 = edge_index[0].reshape(2 * NT, NCH2, CH2)
    dst2 = edge_index[1].reshape(2 * NT, NCH2, CH2)
    parts = _stage_d(src2, dst2, t2).reshape(2, NP, 32)[:, :N]

    return _stage_e(parts, b2)


# speedup vs baseline: 1.9778x; 1.3447x over previous
import jax
import jax.numpy as jnp
from jax import lax
from jax.experimental import pallas as pl
from jax.experimental.pallas import tpu as pltpu
from jax.experimental.pallas import tpu_sc as plsc

N = 10000
E = 160000
F_IN = 256
HID = 64
HEADS = 8
NCLS = 16

RB = 1000
NT = 16
NP = 10240
NPT = NP // NT
ZR = 128
E2 = 163840

CH1 = 128
EPT1 = E2 // NT
NCH1 = EPT1 // CH1
NG1 = NCH1 // 2

CH2 = 40
EPT2 = E // (2 * NT)
NCH2 = EPT2 // CH2


def _lane_take(v, idx16):
    dnums = lax.GatherDimensionNumbers(
        offset_dims=(), collapsed_slice_dims=(0,), start_index_map=(0,))
    return lax.gather(v, idx16[:, None], dnums, (1,),
                      mode=lax.GatherScatterMode.PROMISE_IN_BOUNDS)


def _mm1_body(x_ref, w1_ref, a1s_ref, a1d_ref, h_ref, als_ref, ald_ref):
    h = jnp.dot(x_ref[...], w1_ref[...], preferred_element_type=jnp.float32)
    h_ref[...] = h
    als_ref[...] = jnp.dot(h, a1s_ref[...], preferred_element_type=jnp.float32)
    ald_ref[...] = jnp.dot(h, a1d_ref[...], preferred_element_type=jnp.float32)


def _stage_a(x, W1, A1s, A1d):
    return pl.pallas_call(
        _mm1_body,
        grid=(N // RB,),
        in_specs=[
            pl.BlockSpec((RB, F_IN), lambda r: (r, 0)),
            pl.BlockSpec((F_IN, HEADS * HID), lambda r: (0, 0)),
            pl.BlockSpec((HEADS * HID, HEADS), lambda r: (0, 0)),
            pl.BlockSpec((HEADS * HID, HEADS), lambda r: (0, 0)),
        ],
        out_specs=[
            pl.BlockSpec((RB, HEADS * HID), lambda r: (r, 0)),
            pl.BlockSpec((RB, HEADS), lambda r: (r, 0)),
            pl.BlockSpec((RB, HEADS), lambda r: (r, 0)),
        ],
        out_shape=[
            jax.ShapeDtypeStruct((N, HEADS * HID), jnp.float32),
            jax.ShapeDtypeStruct((N, HEADS), jnp.float32),
            jax.ShapeDtypeStruct((N, HEADS), jnp.float32),
        ],
    )(x, W1, A1s, A1d)


def _sc1_body(src_hbm, dst_hbm, alS_hbm, alD_hbm, hf_hbm, z64_hbm, z16_hbm,
              acc_out, den_out, ex_hbm,
              src_v, dst_v,
              sA, sB, dA, dB, exA, exB, ixA, ixB, hA, hB, mA, mB,
              acc0, den_acc,
              smSA, smSB, smDA, smDB, smNA, smNB, smHA, smHB, smCA, smCB,
              smWA, smWB):
    c = lax.axis_index("c")
    s = lax.axis_index("s")

    BUFS = ((sA, dA, exA, ixA, smSA, smDA, smNA, smWA, hA, mA, smHA, smCA),
            (sB, dB, exB, ixB, smSB, smDB, smNB, smWB, hB, mB, smHB, smCB))

    def zero_acc():
        pltpu.sync_copy(z64_hbm, acc0.at[pl.ds(s * NPT, NPT)])

    zero_acc()
    pltpu.sync_copy(z16_hbm, den_acc.at[pl.ds(s * NPT, NPT)])

    pltpu.sync_copy(src_hbm.at[s], src_v)
    pltpu.sync_copy(dst_hbm.at[s], dst_v)
    plsc.subcore_barrier()

    for p in range(4):
        hh = 4 * c + p
        hh_splat = jnp.full((16,), hh, jnp.int32)

        exbase = c * E2 + s * EPT1

        def issue(j, b):
            sv, dv, exm, ix, smS, smD, smN, smW, hv, mv, smH, smC = BUFS[b]
            if p == 0:
                pltpu.async_copy(alS_hbm.at[src_v.at[j]], sv, smS)
                pltpu.async_copy(alD_hbm.at[dst_v.at[j]], dv, smD)
            else:
                pltpu.async_copy(
                    ex_hbm.at[pl.ds(exbase + j * CH1, CH1)], exm, smS)

            @pl.loop(0, CH1 // 16)
            def _(k):
                ix[pl.ds(k * 16, 16)] = src_v[j, pl.ds(k * 16, 16)] * 8 + hh

            pltpu.async_copy(hf_hbm.at[ix], hv, smH)

        def process(g, j, b):
            sv, dv, exm, ix, smS, smD, smN, smW, hv, mv, smH, smC = BUFS[b]
            if p == 0:
                pltpu.make_async_copy(
                    alS_hbm.at[pl.ds(0, CH1)], sv, smS).wait()
                pltpu.make_async_copy(
                    alD_hbm.at[pl.ds(0, CH1)], dv, smD).wait()

                @pl.when(g > 0)
                def _():
                    pltpu.make_async_copy(
                        alS_hbm.at[pl.ds(0, CH1)], exm, smN).wait()
                    pltpu.make_async_copy(
                        alS_hbm.at[pl.ds(0, CH1)], exm, smW).wait()
            else:
                pltpu.make_async_copy(
                    alS_hbm.at[pl.ds(0, CH1)], exm, smS).wait()

            @pl.when(g > 0)
            def _():
                pltpu.make_async_copy(
                    hf_hbm.at[pl.ds(0, CH1)], mv, smC).wait()

            didx = dst_v.at[j]
            if p == 0:
                @pl.loop(0, CH1)
                def _(e):
                    a = sv[e, pl.ds(0, 16)] + dv[e, pl.ds(0, 16)]
                    exm[e, pl.ds(0, 16)] = jnp.exp(jnp.maximum(a, 0.2 * a))

                pltpu.async_copy(exm, den_acc.at[didx], smN, add=True)
                pltpu.async_copy(
                    exm, ex_hbm.at[pl.ds(exbase + j * CH1, CH1)], smW)

            pltpu.make_async_copy(hf_hbm.at[pl.ds(0, CH1)], hv, smH).wait()

            @pl.loop(0, CH1)
            def _(e):
                exb = _lane_take(exm[e, pl.ds(0, 16)], hh_splat)
                for q in range(HID // 16):
                    mv[e, pl.ds(q * 16, 16)] = hv[e, pl.ds(q * 16, 16)] * exb

            pltpu.async_copy(mv, acc0.at[didx], smC, add=True)

            @pl.when(g < NG1 - 1)
            def _():
                issue(j + 2, b)

        issue(0, 0)
        issue(1, 1)

        @pl.loop(0, NG1)
        def _(g):
            process(g, 2 * g, 0)
            process(g, 2 * g + 1, 1)

        for b in range(2):
            pltpu.make_async_copy(
                hf_hbm.at[pl.ds(0, CH1)], BUFS[b][9], BUFS[b][11]).wait()
        if p == 0:
            for b in range(2):
                pltpu.make_async_copy(
                    alS_hbm.at[pl.ds(0, CH1)], BUFS[b][2], BUFS[b][6]).wait()
                pltpu.make_async_copy(
                    alS_hbm.at[pl.ds(0, CH1)], BUFS[b][2], BUFS[b][7]).wait()

        plsc.subcore_barrier()
        off = s * NPT
        pltpu.sync_copy(acc0.at[pl.ds(off, NPT)],
                        acc_out.at[pl.ds(hh * NP + off, NPT)])
        if p == 0:
            @pl.when(c == 0)
            def _():
                pltpu.sync_copy(den_acc.at[pl.ds(off, NPT)],
                                den_out.at[pl.ds(off, NPT)])
        if p < 3:
            zero_acc()
            plsc.subcore_barrier()


def _stage_b(src2d, dst2d, alS, alD, h_flat):
    mesh = plsc.VectorSubcoreMesh(core_axis_name="c", subcore_axis_name="s")
    kern = pl.kernel(
        _sc1_body,
        mesh=mesh,
        compiler_params=pltpu.CompilerParams(use_tc_tiling_on_sc=False),
        out_type=[
            jax.ShapeDtypeStruct((HEADS * NP, HID), jnp.float32),
            jax.ShapeDtypeStruct((NP, 16), jnp.float32),
            jax.ShapeDtypeStruct((2 * E2, 16), jnp.float32),
        ],
        scratch_types=[
            pltpu.VMEM((NCH1, CH1), jnp.int32),
            pltpu.VMEM((NCH1, CH1), jnp.int32),
            pltpu.VMEM((CH1, 16), jnp.float32),
            pltpu.VMEM((CH1, 16), jnp.float32),
            pltpu.VMEM((CH1, 16), jnp.float32),
            pltpu.VMEM((CH1, 16), jnp.float32),
            pltpu.VMEM((CH1, 16), jnp.float32),
            pltpu.VMEM((CH1, 16), jnp.float32),
            pltpu.VMEM((CH1,), jnp.int32),
            pltpu.VMEM((CH1,), jnp.int32),
            pltpu.VMEM((CH1, HID), jnp.float32),
            pltpu.VMEM((CH1, HID), jnp.float32),
            pltpu.VMEM((CH1, HID), jnp.float32),
            pltpu.VMEM((CH1, HID), jnp.float32),
            pltpu.VMEM_SHARED((NP, HID), jnp.float32),
            pltpu.VMEM_SHARED((NP, 16), jnp.float32),
        ] + [pltpu.SemaphoreType.DMA] * 12,
    )
    return kern(src2d, dst2d, alS, alD, h_flat,
                jnp.zeros((NPT, HID), jnp.float32),
                jnp.zeros((NPT, 16), jnp.float32))


def _mm2_body(acc_ref, den_ref, w2_ref, b1_ref, a2s_ref, a2d_ref, t2_ref):
    h2t = jnp.zeros((RB, NCLS), jnp.float32)
    den = den_ref[...]
    for h in range(HEADS):
        v = acc_ref[h] / (den[:, h][:, None] + 1e-16) + b1_ref[h][None, :]
        v = jnp.where(v > 0, v, jnp.exp(jnp.minimum(v, 0.0)) - 1.0)
        h2t = h2t + jnp.dot(v, w2_ref[h], preferred_element_type=jnp.float32)
    als2 = jnp.sum(h2t * a2s_ref[...], axis=1)
    ald2 = jnp.sum(h2t * a2d_ref[...], axis=1)
    pad = jnp.zeros((RB, 14), jnp.float32)
    t2_ref[...] = jnp.concatenate(
        [h2t, als2[:, None], ald2[:, None], pad], axis=1)


def _stage_c(acc1, den1, W2, b1, a2_src, a2_dst):
    w2r = W2.reshape(HEADS, HID, NCLS)
    b1r = b1.reshape(HEADS, HID)
    return pl.pallas_call(
        _mm2_body,
        grid=(N // RB,),
        in_specs=[
            pl.BlockSpec((HEADS, RB, HID), lambda r: (0, r, 0)),
            pl.BlockSpec((RB, 16), lambda r: (r, 0)),
            pl.BlockSpec((HEADS, HID, NCLS), lambda r: (0, 0, 0)),
            pl.BlockSpec((HEADS, HID), lambda r: (0, 0)),
            pl.BlockSpec((1, NCLS), lambda r: (0, 0)),
            pl.BlockSpec((1, NCLS), lambda r: (0, 0)),
        ],
        out_specs=pl.BlockSpec((RB, 32), lambda r: (r, 0)),
        out_shape=jax.ShapeDtypeStruct((N, 32), jnp.float32),
    )(acc1, den1, w2r, b1r, a2_src, a2_dst)


def _sc2_body(src_hbm, dst_hbm, t2_hbm, acc_out,
              src_v, dst_v, s_rows, d_rows, m_rows, zb, acc2):
    c = lax.axis_index("c")
    s = lax.axis_index("s")
    w = s * 2 + c
    lane = lax.iota(jnp.int32, 16)
    den_mask = lane == 0

    @pl.loop(0, ZR)
    def _(i):
        @pl.loop(0, 2)
        def _(j):
            zb[i, pl.ds(j * 16, 16)] = jnp.zeros((16,), jnp.float32)

    @pl.loop(0, NPT // ZR)
    def _(k):
        pltpu.sync_copy(zb, acc2.at[pl.ds(s * NPT + k * ZR, ZR)])

    pltpu.sync_copy(src_hbm.at[w], src_v)
    pltpu.sync_copy(dst_hbm.at[w], dst_v)
    plsc.subcore_barrier()

    @pl.loop(0, NCH2)
    def _(j):
        sidx = src_v.at[j]
        didx = dst_v.at[j]
        pltpu.sync_copy(t2_hbm.at[sidx], s_rows)
        pltpu.sync_copy(t2_hbm.at[didx], d_rows)

        lane0 = jnp.zeros((16,), jnp.int32)
        lane1 = jnp.ones((16,), jnp.int32)

        @pl.loop(0, CH2)
        def _(e):
            av = (_lane_take(s_rows[e, pl.ds(16, 16)], lane0)
                  + _lane_take(d_rows[e, pl.ds(16, 16)], lane1))
            av = jnp.where(av >= 0, av, 0.2 * av)
            exv = jnp.exp(av)
            m_rows[e, pl.ds(0, 16)] = exv * s_rows[e, pl.ds(0, 16)]
            m_rows[e, pl.ds(16, 16)] = jnp.where(den_mask, exv, 0.0)

        pltpu.sync_copy(m_rows, acc2.at[didx], add=True)

    plsc.subcore_barrier()
    off = s * NPT
    pltpu.sync_copy(acc2.at[pl.ds(off, NPT)],
                    acc_out.at[pl.ds(c * NP + off, NPT)])


def _stage_d(src2d, dst2d, t2):
    mesh = plsc.VectorSubcoreMesh(core_axis_name="c", subcore_axis_name="s")
    kern = pl.kernel(
        _sc2_body,
        mesh=mesh,
        compiler_params=pltpu.CompilerParams(use_tc_tiling_on_sc=False),
        out_type=jax.ShapeDtypeStruct((2 * NP, 32), jnp.float32),
        scratch_types=[
            pltpu.VMEM((NCH2, CH2), jnp.int32),
            pltpu.VMEM((NCH2, CH2), jnp.int32),
            pltpu.VMEM((CH2, 32), jnp.float32),
            pltpu.VMEM((CH2, 32), jnp.float32),
            pltpu.VMEM((CH2, 32), jnp.float32),
            pltpu.VMEM((ZR, 32), jnp.float32),
            pltpu.VMEM_SHARED((NP, 32), jnp.float32),
        ],
    )
    return kern(src2d, dst2d, t2)


def _fin_body(p_ref, b2_ref, o_ref):
    agg = p_ref[0, :, 0:NCLS] + p_ref[1, :, 0:NCLS]
    den = p_ref[0, :, NCLS] + p_ref[1, :, NCLS]
    h2 = agg / (den[:, None] + 1e-16) + b2_ref[...][None, :]
    m = jnp.max(h2, axis=1, keepdims=True)
    sh = h2 - m
    o_ref[...] = sh - jnp.log(jnp.sum(jnp.exp(sh), axis=1, keepdims=True))


def _stage_e(parts, b2):
    return pl.pallas_call(
        _fin_body,
        grid=(N // RB,),
        in_specs=[
            pl.BlockSpec((2, RB, 32), lambda r: (0, r, 0)),
            pl.BlockSpec((NCLS,), lambda r: (0,)),
        ],
        out_specs=pl.BlockSpec((RB, NCLS), lambda r: (r, 0)),
        out_shape=jax.ShapeDtypeStruct((N, NCLS), jnp.float32),
    )(parts, b2)


def kernel(x, edge_index, W1, a1_src, a1_dst, b1, W2, a2_src, a2_dst, b2):
    eye = jnp.eye(HEADS, dtype=jnp.float32)
    A1s = (eye[:, None, :] * a1_src[:, :, None]).reshape(HEADS * HID, HEADS)
    A1d = (eye[:, None, :] * a1_dst[:, :, None]).reshape(HEADS * HID, HEADS)

    h_all, als, ald = _stage_a(x, W1, A1s, A1d)
    alS = jnp.concatenate([als, ald], axis=1)
    alD = jnp.concatenate([ald, als], axis=1)
    h_flat = h_all.reshape(N * HEADS, HID)

    npad = E2 - E
    src_p = jnp.concatenate([edge_index[0], jnp.zeros((npad,), jnp.int32)])
    dst_p = jnp.concatenate(
        [edge_index[1], jnp.full((npad,), NP - 1, jnp.int32)])

    src1 = src_p.reshape(NT, NCH1, CH1)
    dst1 = dst_p.reshape(NT, NCH1, CH1)
    acc1, den1, _excache = _stage_b(src1, dst1, alS, alD, h_flat)
    acc1 = acc1.reshape(HEADS, NP, HID)[:, :N]
    den1 = den1[:N]

    t2 = _stage_c(acc1, den1, W2, b1, a2_src, a2_dst)

    src2 = edge_index[0].reshape(2 * NT, NCH2, CH2)
    dst2 = edge_index[1].reshape(2 * NT, NCH2, CH2)
    parts = _stage_d(src2, dst2, t2).reshape(2, NP, 32)[:, :N]

    return _stage_e(parts, b2)
